# baseline (speedup 1.0000x reference)
import jax
import jax.numpy as jnp
from jax import lax
from jax.experimental import pallas as pl
from jax.experimental.pallas import tpu as pltpu
from jax.experimental.pallas import tpu_sc as plsc

_N = 100000
_E = 1600000
_H = 1024
_D1 = 48
_NC = 2
_NS = 16
_NPAD = 100352
_ROWS = _NPAD // _NS
_ZP = _ROWS // 4
_G = 4
_CH = 10000
_EPT = _E // _NS


def _mesh():
    return plsc.VectorSubcoreMesh(core_axis_name="c", subcore_axis_name="s")


def _degree(dst):
    def body(dst_hbm, cnt_hbm, dstage, ones, zpart, cacc):
        c = lax.axis_index("c")
        s = lax.axis_index("s")

        def _f1(j, carry):
            ones[pl.ds(j * 16, 16)] = jnp.full((16,), jnp.float32(1.0))
            return carry
        lax.fori_loop(0, _CH // 16, _f1, 0)

        def _f0(j, carry):
            zpart[pl.ds(j * 16, 16)] = jnp.zeros((16,), jnp.float32)
            return carry
        lax.fori_loop(0, _ROWS // 16, _f0, 0)

        @pl.when(c == 0)
        def _():
            pltpu.sync_copy(zpart, cacc.at[pl.ds(s * _ROWS, _ROWS)])
            plsc.subcore_barrier()

            def chunk_body(k, carry):
                base = s * _EPT + k * _CH
                pltpu.sync_copy(dst_hbm.at[pl.ds(base, _CH)], dstage)
                pltpu.sync_copy(ones, cacc.at[dstage], add=True)
                return carry
            lax.fori_loop(0, _EPT // _CH, chunk_body, 0)

            plsc.subcore_barrier()
            pltpu.sync_copy(cacc.at[pl.ds(s * _ROWS, _ROWS)],
                            cnt_hbm.at[pl.ds(s * _ROWS, _ROWS)])

    f = pl.kernel(
        body,
        compiler_params=pltpu.CompilerParams(needs_layout_passes=False),
        out_type=jax.ShapeDtypeStruct((_NPAD,), jnp.float32),
        mesh=_mesh(),
        scratch_types=[
            pltpu.VMEM((_CH,), jnp.int32),
            pltpu.VMEM((_CH,), jnp.float32),
            pltpu.VMEM((_ROWS,), jnp.float32),
            pltpu.VMEM_SHARED((_NPAD,), jnp.float32),
        ],
    )
    return f(dst)


def _segment_sum_t(src, dst, x_t, d):
    ngrp = d // (_G * _NC)

    def body(src_hbm, dst_hbm, x_hbm, seg_hbm,
             ss0, ds0, zpart,
             v00, v01, v02, v03,
             cb0, cb1, cb2, cb3, ac0, ac1, ac2, ac3,
             gs0, gs1, gs2, gs3):
        c = lax.axis_index("c")
        s = lax.axis_index("s")
        sstage = (ss0,)
        dstage = (ds0,)
        vals = ((v00, v01, v02, v03),)
        cols = (cb0, cb1, cb2, cb3)
        accs = (ac0, ac1, ac2, ac3)
        gsem = (gs0, gs1, gs2, gs3)

        def _f0(j, carry):
            zpart[pl.ds(j * 16, 16)] = jnp.zeros((16,), jnp.float32)
            return carry
        lax.fori_loop(0, _ZP // 16, _f0, 0)

        def grp_body(grp, carry):
            cbase = (grp * _NC + c) * _G
            for g in range(_G):
                pltpu.sync_copy(
                    x_hbm.at[cbase + g, pl.ds(s * _ROWS, _ROWS)],
                    cols[g].at[pl.ds(s * _ROWS, _ROWS)])
                for q in range(4):
                    pltpu.sync_copy(zpart, accs[g].at[
                        pl.ds(s * _ROWS + q * _ZP, _ZP)])
            plsc.subcore_barrier()

            def chunk_body(k, carry2):
                base = s * _EPT + k * _CH
                pltpu.sync_copy(src_hbm.at[pl.ds(base, _CH)], sstage[0])
                pltpu.sync_copy(dst_hbm.at[pl.ds(base, _CH)], dstage[0])
                gd = [pltpu.async_copy(cols[g].at[sstage[0]], vals[0][g],
                                       gsem[g]) for g in range(_G)]
                for g in range(_G):
                    gd[g].wait()
                    pltpu.sync_copy(vals[0][g], accs[g].at[dstage[0]],
                                    add=True)
                return carry2
            lax.fori_loop(0, _EPT // _CH, chunk_body, 0)

            plsc.subcore_barrier()
            for g in range(_G):
                pltpu.sync_copy(accs[g].at[pl.ds(s * _ROWS, _ROWS)],
                                seg_hbm.at[cbase + g, pl.ds(s * _ROWS, _ROWS)])
            plsc.subcore_barrier()
            return carry
        lax.fori_loop(0, ngrp, grp_body, 0)

    f = pl.kernel(
        body,
        compiler_params=pltpu.CompilerParams(needs_layout_passes=False),
        out_type=jax.ShapeDtypeStruct((d, _NPAD), jnp.float32),
        mesh=_mesh(),
        scratch_types=(
            [pltpu.VMEM((_CH,), jnp.int32)] * 2
            + [pltpu.VMEM((_ZP,), jnp.float32)]
            + [pltpu.VMEM((_CH,), jnp.float32)] * 4
            + [pltpu.VMEM_SHARED((_NPAD,), jnp.float32)] * 8
            + [pltpu.SemaphoreType.DMA] * 4
        ),
    )
    return f(src, dst, x_t)


_BN = 1024


def _tc_layer_t(seg_t, cnt1r, x_t, wl, blc, wr):
    d = x_t.shape[0]
    h = wl.shape[0]

    def tc_body(seg_ref, cnt_ref, x_ref, wl_ref, bl_ref, wr_ref, o_ref):
        inv = 1.0 / jnp.maximum(cnt_ref[...], 1.0)
        mean_t = seg_ref[...] * inv
        acc = lax.dot_general(wl_ref[...], mean_t, (((1,), (0,)), ((), ())),
                              preferred_element_type=jnp.float32)
        acc = acc + lax.dot_general(wr_ref[...], x_ref[...],
                                    (((1,), (0,)), ((), ())),
                                    preferred_element_type=jnp.float32)
        acc = acc + bl_ref[...]
        o_ref[...] = jnp.maximum(acc, 0.0)

    return pl.pallas_call(
        tc_body,
        grid=(_NPAD // _BN,),
        in_specs=[
            pl.BlockSpec((d, _BN), lambda i: (0, i)),
            pl.BlockSpec((1, _BN), lambda i: (0, i)),
            pl.BlockSpec((d, _BN), lambda i: (0, i)),
            pl.BlockSpec((h, d), lambda i: (0, 0)),
            pl.BlockSpec((h, 1), lambda i: (0, 0)),
            pl.BlockSpec((h, d), lambda i: (0, 0)),
        ],
        out_specs=pl.BlockSpec((h, _BN), lambda i: (0, i)),
        out_shape=jax.ShapeDtypeStruct((h, _NPAD), jnp.float32),
    )(seg_t, cnt1r, x_t, wl, blc, wr)


def _tc_layer_final_t(seg_t, cnt1r, x_t, wl, blc, wr, wlin, blin2d):
    d = x_t.shape[0]
    h = wl.shape[0]

    def tc_body(seg_ref, cnt_ref, x_ref, wl_ref, bl_ref, wr_ref, wlin_ref,
                blin_ref, o_ref):
        inv = 1.0 / jnp.maximum(cnt_ref[...], 1.0)
        mean_t = seg_ref[...] * inv
        acc = lax.dot_general(wl_ref[...], mean_t, (((1,), (0,)), ((), ())),
                              preferred_element_type=jnp.float32)
        acc = acc + lax.dot_general(wr_ref[...], x_ref[...],
                                    (((1,), (0,)), ((), ())),
                                    preferred_element_type=jnp.float32)
        acc = acc + bl_ref[...]
        hrelu = jnp.maximum(acc, 0.0)
        o_ref[...] = lax.dot_general(hrelu, wlin_ref[...],
                                     (((0,), (1,)), ((), ())),
                                     preferred_element_type=jnp.float32) \
            + blin_ref[...]

    return pl.pallas_call(
        tc_body,
        grid=(_NPAD // _BN,),
        in_specs=[
            pl.BlockSpec((d, _BN), lambda i: (0, i)),
            pl.BlockSpec((1, _BN), lambda i: (0, i)),
            pl.BlockSpec((d, _BN), lambda i: (0, i)),
            pl.BlockSpec((h, d), lambda i: (0, 0)),
            pl.BlockSpec((h, 1), lambda i: (0, 0)),
            pl.BlockSpec((h, d), lambda i: (0, 0)),
            pl.BlockSpec((1, h), lambda i: (0, 0)),
            pl.BlockSpec((1, 1), lambda i: (0, 0)),
        ],
        out_specs=pl.BlockSpec((_BN, 1), lambda i: (i, 0)),
        out_shape=jax.ShapeDtypeStruct((_NPAD, 1), jnp.float32),
    )(seg_t, cnt1r, x_t, wl, blc, wr, wlin, blin2d)


def kernel(feature, edge_index, W1l, b1l, W1r, W2l, b2l, W2r, W3l, b3l, W3r,
           Wlin, blin):
    src = edge_index[0]
    dst = edge_index[1]

    feat_t = jnp.pad(feature, ((0, _NPAD - _N), (0, _D1 - feature.shape[1]))).T
    w1l = jnp.pad(W1l, ((0, 0), (0, _D1 - W1l.shape[1])))
    w1r = jnp.pad(W1r, ((0, 0), (0, _D1 - W1r.shape[1])))

    cnt1r = _degree(dst).reshape(1, _NPAD)

    b1 = b1l.reshape(_H, 1)
    b2 = b2l.reshape(_H, 1)
    b3 = b3l.reshape(_H, 1)
    blin2d = blin.reshape(1, 1)

    seg1 = _segment_sum_t(src, dst, feat_t, _D1)
    h1 = _tc_layer_t(seg1, cnt1r, feat_t, w1l, b1, w1r)
    seg2 = _segment_sum_t(src, dst, h1, _H)
    h2 = _tc_layer_t(seg2, cnt1r, h1, W2l, b2, W2r)
    seg3 = _segment_sum_t(src, dst, h2, _H)
    out = _tc_layer_final_t(seg3, cnt1r, h2, W3l, b3, W3r, Wlin, blin2d)
    return out[:_N]


# speedup vs baseline: 1.0023x; 1.0023x over previous
import jax
import jax.numpy as jnp
from jax import lax
from jax.experimental import pallas as pl
from jax.experimental.pallas import tpu as pltpu
from jax.experimental.pallas import tpu_sc as plsc

_N = 100000
_E = 1600000
_H = 1024
_D1 = 48
_NC = 2
_NS = 16
_NPAD = 100352
_ROWS = _NPAD // _NS
_G = 4
_CH = 10000
_EPT = _E // _NS


def _mesh():
    return plsc.VectorSubcoreMesh(core_axis_name="c", subcore_axis_name="s")


def _degree(dst):
    def body(dst_hbm, cnt_hbm, dstage, ones, zpart, cacc):
        c = lax.axis_index("c")
        s = lax.axis_index("s")

        def _f1(j, carry):
            ones[pl.ds(j * 16, 16)] = jnp.full((16,), jnp.float32(1.0))
            return carry
        lax.fori_loop(0, _CH // 16, _f1, 0)

        def _f0(j, carry):
            zpart[pl.ds(j * 16, 16)] = jnp.zeros((16,), jnp.float32)
            return carry
        lax.fori_loop(0, _ROWS // 16, _f0, 0)

        @pl.when(c == 0)
        def _():
            pltpu.sync_copy(zpart, cacc.at[pl.ds(s * _ROWS, _ROWS)])
            plsc.subcore_barrier()

            def chunk_body(k, carry):
                base = s * _EPT + k * _CH
                pltpu.sync_copy(dst_hbm.at[pl.ds(base, _CH)], dstage)
                pltpu.sync_copy(ones, cacc.at[dstage], add=True)
                return carry
            lax.fori_loop(0, _EPT // _CH, chunk_body, 0)

            plsc.subcore_barrier()
            pltpu.sync_copy(cacc.at[pl.ds(s * _ROWS, _ROWS)],
                            cnt_hbm.at[pl.ds(s * _ROWS, _ROWS)])

    f = pl.kernel(
        body,
        compiler_params=pltpu.CompilerParams(needs_layout_passes=False),
        out_type=jax.ShapeDtypeStruct((_NPAD,), jnp.float32),
        mesh=_mesh(),
        scratch_types=[
            pltpu.VMEM((_CH,), jnp.int32),
            pltpu.VMEM((_CH,), jnp.float32),
            pltpu.VMEM((_ROWS,), jnp.float32),
            pltpu.VMEM_SHARED((_NPAD,), jnp.float32),
        ],
    )
    return f(dst)


def _segment_sum_t(src, dst, x_t, d):
    ngrp = d // (_G * _NC)

    def body(src_hbm, dst_hbm, x_hbm, seg_hbm,
             ss0, ds0, zpart,
             v00, v01, v02, v03,
             cb0, cb1, cb2, cb3, ac0, ac1, ac2, ac3,
             gs0, gs1, gs2, gs3):
        c = lax.axis_index("c")
        s = lax.axis_index("s")
        sstage = (ss0,)
        dstage = (ds0,)
        vals = ((v00, v01, v02, v03),)
        cols = (cb0, cb1, cb2, cb3)
        accs = (ac0, ac1, ac2, ac3)
        gsem = (gs0, gs1, gs2, gs3)

        def _f0(j, carry):
            zpart[pl.ds(j * 16, 16)] = jnp.zeros((16,), jnp.float32)
            return carry
        lax.fori_loop(0, _ROWS // 16, _f0, 0)

        def grp_body(grp, carry):
            cbase = (grp * _NC + c) * _G
            for g in range(_G):
                pltpu.sync_copy(
                    x_hbm.at[cbase + g, pl.ds(s * _ROWS, _ROWS)],
                    cols[g].at[pl.ds(s * _ROWS, _ROWS)])
                pltpu.sync_copy(zpart, accs[g].at[pl.ds(s * _ROWS, _ROWS)])
            plsc.subcore_barrier()

            def chunk_body(k, carry2):
                base = s * _EPT + k * _CH
                pltpu.sync_copy(src_hbm.at[pl.ds(base, _CH)], sstage[0])
                pltpu.sync_copy(dst_hbm.at[pl.ds(base, _CH)], dstage[0])
                gd = [pltpu.async_copy(cols[g].at[sstage[0]], vals[0][g],
                                       gsem[g]) for g in range(_G)]
                for g in range(_G):
                    gd[g].wait()
                    pltpu.sync_copy(vals[0][g], accs[g].at[dstage[0]],
                                    add=True)
                return carry2
            lax.fori_loop(0, _EPT // _CH, chunk_body, 0)

            plsc.subcore_barrier()
            for g in range(_G):
                pltpu.sync_copy(accs[g].at[pl.ds(s * _ROWS, _ROWS)],
                                seg_hbm.at[cbase + g, pl.ds(s * _ROWS, _ROWS)])
            plsc.subcore_barrier()
            return carry
        lax.fori_loop(0, ngrp, grp_body, 0)

    f = pl.kernel(
        body,
        compiler_params=pltpu.CompilerParams(needs_layout_passes=False),
        out_type=jax.ShapeDtypeStruct((d, _NPAD), jnp.float32),
        mesh=_mesh(),
        scratch_types=(
            [pltpu.VMEM((_CH,), jnp.int32)] * 2
            + [pltpu.VMEM((_ROWS,), jnp.float32)]
            + [pltpu.VMEM((_CH,), jnp.float32)] * 4
            + [pltpu.VMEM_SHARED((_NPAD,), jnp.float32)] * 8
            + [pltpu.SemaphoreType.DMA] * 4
        ),
    )
    return f(src, dst, x_t)


_BN = 1024


def _tc_layer_t(seg_t, cnt1r, x_t, wl, blc, wr):
    d = x_t.shape[0]
    h = wl.shape[0]

    def tc_body(seg_ref, cnt_ref, x_ref, wl_ref, bl_ref, wr_ref, o_ref):
        inv = 1.0 / jnp.maximum(cnt_ref[...], 1.0)
        mean_t = seg_ref[...] * inv
        acc = lax.dot_general(wl_ref[...], mean_t, (((1,), (0,)), ((), ())),
                              preferred_element_type=jnp.float32)
        acc = acc + lax.dot_general(wr_ref[...], x_ref[...],
                                    (((1,), (0,)), ((), ())),
                                    preferred_element_type=jnp.float32)
        acc = acc + bl_ref[...]
        o_ref[...] = jnp.maximum(acc, 0.0)

    return pl.pallas_call(
        tc_body,
        grid=(_NPAD // _BN,),
        in_specs=[
            pl.BlockSpec((d, _BN), lambda i: (0, i)),
            pl.BlockSpec((1, _BN), lambda i: (0, i)),
            pl.BlockSpec((d, _BN), lambda i: (0, i)),
            pl.BlockSpec((h, d), lambda i: (0, 0)),
            pl.BlockSpec((h, 1), lambda i: (0, 0)),
            pl.BlockSpec((h, d), lambda i: (0, 0)),
        ],
        out_specs=pl.BlockSpec((h, _BN), lambda i: (0, i)),
        out_shape=jax.ShapeDtypeStruct((h, _NPAD), jnp.float32),
    )(seg_t, cnt1r, x_t, wl, blc, wr)


def _tc_layer_final_t(seg_t, cnt1r, x_t, wl, blc, wr, wlin, blin2d):
    d = x_t.shape[0]
    h = wl.shape[0]

    def tc_body(seg_ref, cnt_ref, x_ref, wl_ref, bl_ref, wr_ref, wlin_ref,
                blin_ref, o_ref):
        inv = 1.0 / jnp.maximum(cnt_ref[...], 1.0)
        mean_t = seg_ref[...] * inv
        acc = lax.dot_general(wl_ref[...], mean_t, (((1,), (0,)), ((), ())),
                              preferred_element_type=jnp.float32)
        acc = acc + lax.dot_general(wr_ref[...], x_ref[...],
                                    (((1,), (0,)), ((), ())),
                                    preferred_element_type=jnp.float32)
        acc = acc + bl_ref[...]
        hrelu = jnp.maximum(acc, 0.0)
        o_ref[...] = lax.dot_general(hrelu, wlin_ref[...],
                                     (((0,), (1,)), ((), ())),
                                     preferred_element_type=jnp.float32) \
            + blin_ref[...]

    return pl.pallas_call(
        tc_body,
        grid=(_NPAD // _BN,),
        in_specs=[
            pl.BlockSpec((d, _BN), lambda i: (0, i)),
            pl.BlockSpec((1, _BN), lambda i: (0, i)),
            pl.BlockSpec((d, _BN), lambda i: (0, i)),
            pl.BlockSpec((h, d), lambda i: (0, 0)),
            pl.BlockSpec((h, 1), lambda i: (0, 0)),
            pl.BlockSpec((h, d), lambda i: (0, 0)),
            pl.BlockSpec((1, h), lambda i: (0, 0)),
            pl.BlockSpec((1, 1), lambda i: (0, 0)),
        ],
        out_specs=pl.BlockSpec((_BN, 1), lambda i: (i, 0)),
        out_shape=jax.ShapeDtypeStruct((_NPAD, 1), jnp.float32),
    )(seg_t, cnt1r, x_t, wl, blc, wr, wlin, blin2d)


def kernel(feature, edge_index, W1l, b1l, W1r, W2l, b2l, W2r, W3l, b3l, W3r,
           Wlin, blin):
    src = edge_index[0]
    dst = edge_index[1]

    feat_t = jnp.pad(feature, ((0, _NPAD - _N), (0, _D1 - feature.shape[1]))).T
    w1l = jnp.pad(W1l, ((0, 0), (0, _D1 - W1l.shape[1])))
    w1r = jnp.pad(W1r, ((0, 0), (0, _D1 - W1r.shape[1])))

    cnt1r = _degree(dst).reshape(1, _NPAD)

    b1 = b1l.reshape(_H, 1)
    b2 = b2l.reshape(_H, 1)
    b3 = b3l.reshape(_H, 1)
    blin2d = blin.reshape(1, 1)

    seg1 = _segment_sum_t(src, dst, feat_t, _D1)
    h1 = _tc_layer_t(seg1, cnt1r, feat_t, w1l, b1, w1r)
    seg2 = _segment_sum_t(src, dst, h1, _H)
    h2 = _tc_layer_t(seg2, cnt1r, h1, W2l, b2, W2r)
    seg3 = _segment_sum_t(src, dst, h2, _H)
    out = _tc_layer_final_t(seg3, cnt1r, h2, W3l, b3, W3r, Wlin, blin2d)
    return out[:_N]
